# Initial kernel scaffold; baseline (speedup 1.0000x reference)
import jax
import jax.numpy as jnp
from jax.experimental import pallas as pl


def kernel(batch_user_ids, batch_movie_ids, item_bin_ids, user_time_dev, user_emb, item_emb, user_bias, item_bias, user_dev_param, item_bin_param):
    raise NotImplementedError("write your pallas kernel here")


# speedup vs baseline: 1.3430x; 1.3430x over previous
import dataclasses
import functools

import jax
import jax.numpy as jnp
from jax import lax
from jax.experimental import pallas as pl
from jax.experimental.pallas import tpu as pltpu
from jax.experimental.pallas import tpu_sc as plsc

NC = 2
NS = 16
L = 16
NW = NC * NS
BATCH = 16384
D = 128
CHUNK = 128
PER_W = BATCH // NW
N_CHUNK = PER_W // CHUNK
GMEAN = 3.6


def _body(uid_hbm, mid_hbm, bin_hbm, utd_hbm,
          uemb_hbm, iemb_hbm, ub_hbm, ib_hbm, ud_hbm, bp_hbm,
          out_hbm,
          uidx_v, midx_v, bidx_v, utd_v, binp_v,
          u_v, i_v, ub_v, ud_v, ib_v, out_v, sem):
    wid = lax.axis_index("s") * NC + lax.axis_index("c")

    cps = [
        pltpu.async_copy(uid_hbm.at[wid], uidx_v, sem),
        pltpu.async_copy(mid_hbm.at[wid], midx_v, sem),
        pltpu.async_copy(bin_hbm.at[wid], bidx_v, sem),
        pltpu.async_copy(utd_hbm.at[wid], utd_v, sem),
        pltpu.async_copy(bp_hbm, binp_v, sem),
    ]
    for cp in cps:
        cp.wait()

    lane = lax.iota(jnp.int32, L)

    for c in range(N_CHUNK):
        cps = [
            pltpu.async_copy(uemb_hbm.at[uidx_v.at[c]], u_v, sem),
            pltpu.async_copy(iemb_hbm.at[midx_v.at[c]], i_v, sem),
            pltpu.async_copy(ub_hbm.at[uidx_v.at[c]], ub_v, sem),
            pltpu.async_copy(ud_hbm.at[uidx_v.at[c]], ud_v, sem),
            pltpu.async_copy(ib_hbm.at[midx_v.at[c]], ib_v, sem),
        ]
        for cp in cps:
            cp.wait()

        @pl.loop(0, CHUNK // L)
        def _(g):
            base = g * L
            dots = jnp.zeros((L,), jnp.float32)
            for rl in range(L):
                row = base + rl
                acc = u_v[row, pl.ds(0, L)] * i_v[row, pl.ds(0, L)]
                for d in range(1, D // L):
                    acc = acc + (u_v[row, pl.ds(d * L, L)] *
                                 i_v[row, pl.ds(d * L, L)])
                dots = jnp.where(lane == rl, jnp.sum(acc), dots)
            sl = pl.ds(base, L)
            bins = bidx_v[c, sl]
            bp = plsc.load_gather(binp_v, [bins])
            res = (dots + ub_v[sl] + ud_v[sl] * utd_v[c, sl]
                   + ib_v[sl] + bp + GMEAN)
            out_v[c, sl] = res

    pltpu.sync_copy(out_v, out_hbm.at[wid])


def kernel(batch_user_ids, batch_movie_ids, item_bin_ids, user_time_dev,
           user_emb, item_emb, user_bias, item_bias, user_dev_param,
           item_bin_param):
    uid = batch_user_ids.astype(jnp.int32).reshape(NW, N_CHUNK, CHUNK)
    mid = batch_movie_ids.astype(jnp.int32).reshape(NW, N_CHUNK, CHUNK)
    bins = item_bin_ids.astype(jnp.int32).reshape(NW, N_CHUNK, CHUNK)
    utd = user_time_dev.reshape(NW, N_CHUNK, CHUNK)
    ub = user_bias.reshape(-1)
    ib = item_bias.reshape(-1)
    ud = user_dev_param.reshape(-1)
    bp = jnp.pad(item_bin_param.reshape(-1), (0, 1))

    mesh = plsc.VectorSubcoreMesh(core_axis_name="c", subcore_axis_name="s")
    cp = pltpu.CompilerParams()
    if "needs_layout_passes" in pltpu.CompilerParams.__dataclass_fields__:
        cp = dataclasses.replace(cp, needs_layout_passes=False)
    run = pl.kernel(
        _body,
        out_type=jax.ShapeDtypeStruct((NW, N_CHUNK, CHUNK), jnp.float32),
        mesh=mesh,
        scratch_types=[
            pltpu.VMEM((N_CHUNK, CHUNK), jnp.int32),
            pltpu.VMEM((N_CHUNK, CHUNK), jnp.int32),
            pltpu.VMEM((N_CHUNK, CHUNK), jnp.int32),
            pltpu.VMEM((N_CHUNK, CHUNK), jnp.float32),
            pltpu.VMEM((32,), jnp.float32),
            pltpu.VMEM((CHUNK, D), jnp.float32),
            pltpu.VMEM((CHUNK, D), jnp.float32),
            pltpu.VMEM((CHUNK,), jnp.float32),
            pltpu.VMEM((CHUNK,), jnp.float32),
            pltpu.VMEM((CHUNK,), jnp.float32),
            pltpu.VMEM((N_CHUNK, CHUNK), jnp.float32),
            pltpu.SemaphoreType.DMA,
        ],
        compiler_params=cp,
    )
    out = run(uid, mid, bins, utd, user_emb, item_emb, ub, ib, ud, bp)
    return out.reshape(-1)


# speedup vs baseline: 2.0048x; 1.4928x over previous
import dataclasses
import functools

import jax
import jax.numpy as jnp
from jax import lax
from jax.experimental import pallas as pl
from jax.experimental.pallas import tpu as pltpu
from jax.experimental.pallas import tpu_sc as plsc

NC = 2
NS = 16
L = 16
NW = NC * NS
BATCH = 16384
D = 128
CHUNK = 128
PER_W = BATCH // NW
N_CHUNK = PER_W // CHUNK
GMEAN = 3.6


def _body(uid_hbm, mid_hbm, bin_hbm, utd_hbm,
          uemb_hbm, iemb_hbm, ub_hbm, ib_hbm, ud_hbm, bp_hbm,
          out_hbm,
          uidx_v, midx_v, bidx_v, utd_v, binp_v,
          u_v0, i_v0, ub_v0, ud_v0, ib_v0,
          u_v1, i_v1, ub_v1, ud_v1, ib_v1,
          out_v, sem0, sem1):
    wid = lax.axis_index("s") * NC + lax.axis_index("c")
    bufs = ((u_v0, i_v0, ub_v0, ud_v0, ib_v0),
            (u_v1, i_v1, ub_v1, ud_v1, ib_v1))
    sems = (sem0, sem1)

    cps = [
        pltpu.async_copy(uid_hbm.at[wid], uidx_v, sem0),
        pltpu.async_copy(mid_hbm.at[wid], midx_v, sem0),
        pltpu.async_copy(bin_hbm.at[wid], bidx_v, sem0),
        pltpu.async_copy(utd_hbm.at[wid], utd_v, sem0),
        pltpu.async_copy(bp_hbm, binp_v, sem0),
    ]
    for cp in cps:
        cp.wait()

    lane = lax.iota(jnp.int32, L)

    def fire(c):
        u_v, i_v, ub_v, ud_v, ib_v = bufs[c % 2]
        sem = sems[c % 2]
        return [
            pltpu.async_copy(uemb_hbm.at[uidx_v.at[c]], u_v, sem),
            pltpu.async_copy(iemb_hbm.at[midx_v.at[c]], i_v, sem),
            pltpu.async_copy(ub_hbm.at[uidx_v.at[c]], ub_v, sem),
            pltpu.async_copy(ud_hbm.at[uidx_v.at[c]], ud_v, sem),
            pltpu.async_copy(ib_hbm.at[midx_v.at[c]], ib_v, sem),
        ]

    inflight = {0: fire(0)}
    for c in range(N_CHUNK):
        if c + 1 < N_CHUNK:
            inflight[c + 1] = fire(c + 1)
        for cp in inflight.pop(c):
            cp.wait()
        u_v, i_v, ub_v, ud_v, ib_v = bufs[c % 2]

        @pl.loop(0, CHUNK // L)
        def _(g):
            base = g * L

            def sub(s, dots):
                for rl in range(4):
                    row = base + s * 4 + rl
                    ps = [u_v[row, pl.ds(d * L, L)] *
                          i_v[row, pl.ds(d * L, L)] for d in range(D // L)]
                    while len(ps) > 1:
                        ps = [a + b for a, b in zip(ps[::2], ps[1::2])]
                    dots = jnp.where(lane == s * 4 + rl, jnp.sum(ps[0]),
                                     dots)
                return dots

            dots = lax.fori_loop(0, 4, sub, jnp.zeros((L,), jnp.float32))
            sl = pl.ds(base, L)
            bins = bidx_v[c, sl]
            bp = plsc.load_gather(binp_v, [bins])
            res = (dots + ub_v[sl] + ud_v[sl] * utd_v[c, sl]
                   + ib_v[sl] + bp + GMEAN)
            out_v[c, sl] = res

    pltpu.sync_copy(out_v, out_hbm.at[wid])


def kernel(batch_user_ids, batch_movie_ids, item_bin_ids, user_time_dev,
           user_emb, item_emb, user_bias, item_bias, user_dev_param,
           item_bin_param):
    uid = batch_user_ids.astype(jnp.int32).reshape(NW, N_CHUNK, CHUNK)
    mid = batch_movie_ids.astype(jnp.int32).reshape(NW, N_CHUNK, CHUNK)
    bins = item_bin_ids.astype(jnp.int32).reshape(NW, N_CHUNK, CHUNK)
    utd = user_time_dev.reshape(NW, N_CHUNK, CHUNK)
    ub = user_bias.reshape(-1)
    ib = item_bias.reshape(-1)
    ud = user_dev_param.reshape(-1)
    bp = jnp.pad(item_bin_param.reshape(-1), (0, 1))

    mesh = plsc.VectorSubcoreMesh(core_axis_name="c", subcore_axis_name="s")
    cp = pltpu.CompilerParams()
    if "needs_layout_passes" in pltpu.CompilerParams.__dataclass_fields__:
        cp = dataclasses.replace(cp, needs_layout_passes=False)
    run = pl.kernel(
        _body,
        out_type=jax.ShapeDtypeStruct((NW, N_CHUNK, CHUNK), jnp.float32),
        mesh=mesh,
        scratch_types=[
            pltpu.VMEM((N_CHUNK, CHUNK), jnp.int32),
            pltpu.VMEM((N_CHUNK, CHUNK), jnp.int32),
            pltpu.VMEM((N_CHUNK, CHUNK), jnp.int32),
            pltpu.VMEM((N_CHUNK, CHUNK), jnp.float32),
            pltpu.VMEM((32,), jnp.float32),
            pltpu.VMEM((CHUNK, D), jnp.float32),
            pltpu.VMEM((CHUNK, D), jnp.float32),
            pltpu.VMEM((CHUNK,), jnp.float32),
            pltpu.VMEM((CHUNK,), jnp.float32),
            pltpu.VMEM((CHUNK,), jnp.float32),
            pltpu.VMEM((CHUNK, D), jnp.float32),
            pltpu.VMEM((CHUNK, D), jnp.float32),
            pltpu.VMEM((CHUNK,), jnp.float32),
            pltpu.VMEM((CHUNK,), jnp.float32),
            pltpu.VMEM((CHUNK,), jnp.float32),
            pltpu.VMEM((N_CHUNK, CHUNK), jnp.float32),
            pltpu.SemaphoreType.DMA,
            pltpu.SemaphoreType.DMA,
        ],
        compiler_params=cp,
    )
    out = run(uid, mid, bins, utd, user_emb, item_emb, ub, ib, ud, bp)
    return out.reshape(-1)
